# baseline (speedup 1.0000x reference)
import functools

import jax
import jax.numpy as jnp
from jax import lax
from jax.experimental import pallas as pl
from jax.experimental.pallas import tpu as pltpu
from jax.experimental.pallas import tpu_sc as plsc

K = 12
D = 64
N_ROWS = 19
B = 1024
L = 200

NC, NS = 2, 16
NW = NC * NS
L_UNITS_MAX = (L + NW - 1) // NW + 1
UNIT = 2 * 8 * 8 * 128
SLAB = D * B


def _table_body(vc_ref, cc_ref, wv_ref, wc_ref, out_ref):
    zero = jnp.zeros((1, K), jnp.float32)
    aug_v = jnp.concatenate([zero, vc_ref[...]], axis=0)
    aug_c = jnp.concatenate([zero, cc_ref[...]], axis=0)
    tv = jnp.dot(aug_v, wv_ref[...], preferred_element_type=jnp.float32,
                 precision=jax.lax.Precision.HIGHEST)
    tc = jnp.dot(aug_c, wc_ref[...], preferred_element_type=jnp.float32,
                 precision=jax.lax.Precision.HIGHEST)
    out_ref[...] = jnp.concatenate([tv, tc], axis=-1)


def _build_table(vc, cc, wv, wc):
    return pl.pallas_call(
        _table_body,
        out_shape=jax.ShapeDtypeStruct((N_ROWS, D), jnp.float32),
    )(vc, cc, wv, wc)


def _sc_body(table_hbm, idx_hbm, out_hbm,
             table_v, icol, svtab, stag_a, stag_b, sem_a, sem_b, isem):
    w = lax.axis_index("s") * NC + lax.axis_index("c")
    pltpu.sync_copy(table_hbm, table_v)

    iota = lax.iota(jnp.int32, 16)
    for d in range(16):
        cr = (iota + d) & 15
        svtab[pl.ds(d * 16, 16)] = (cr >> 3) * 8192 + (cr & 7) * 128 + iota

    lo = w * 25
    hi = lo + 25
    stags = (stag_a, stag_b)
    sems = (sem_a, sem_b)

    l0 = lo >> 2
    for bb in range(8):
        pltpu.async_copy(
            idx_hbm.at[l0 >> 3, bb, l0 & 7],
            icol.at[l0 & 1, pl.ds(bb * 128, 128)],
            isem,
        )

    def _unit(u, carry):
        l = u >> 2
        cq = u & 3
        isel = l & 1

        @pl.when((cq == 0) | (u == lo))
        def _drain_icol():
            for bb in range(8):
                pltpu.make_async_copy(
                    idx_hbm.at[0, 0, 0],
                    icol.at[isel, pl.ds(bb * 128, 128)],
                    isem,
                ).wait()

        @pl.when(((cq == 3) | (u == hi - 1)) & (u + 1 < hi))
        def _prefetch():
            nxt = l + 1
            for bb in range(8):
                pltpu.async_copy(
                    idx_hbm.at[nxt >> 3, bb, nxt & 7],
                    icol.at[1 - isel, pl.ds(bb * 128, 128)],
                    isem,
                )

        for par in range(2):
            stag = stags[par]
            sem = sems[par]

            @pl.when((u & 1) == par)
            def _do(_stag=stag, _sem=sem, _par=par):
                @pl.when(u >= lo + 2)
                def _drain():
                    pltpu.make_async_copy(out_hbm.at[0], _stag, _sem).wait()

                @plsc.parallel_loop(0, 64, unroll=4)
                def _group(g, _stag=_stag):
                    v = icol[isel, pl.ds(g * 16, 16)]
                    v64 = v * D
                    goff = (g >> 3) * 1024 + (g & 7) * 16
                    tb = cq * 16
                    dg = iota
                    for d in range(16):
                        vals = plsc.load_gather(
                            table_v.at[pl.ds(tb, (N_ROWS - 1) * D + 16)],
                            [v64 + dg],
                        )
                        sv = svtab[pl.ds(d * 16, 16)]
                        plsc.store_scatter(
                            _stag.at[pl.ds(goff, 8192 + 896 + 16)], [sv], vals
                        )
                        dg = (dg + 1) & 15

                pltpu.async_copy(_stag, out_hbm.at[u], _sem)
        return carry

    lax.fori_loop(lo, hi, _unit, 0)
    for par in range(2):
        pltpu.make_async_copy(out_hbm.at[0], stags[par], sems[par]).wait()


_gather_rows = functools.partial(
    pl.kernel,
    out_type=jax.ShapeDtypeStruct((L * 4, UNIT), jnp.float32),
    mesh=plsc.VectorSubcoreMesh(core_axis_name="c", subcore_axis_name="s"),
    scratch_types=[
        pltpu.VMEM((N_ROWS * D,), jnp.float32),
        pltpu.VMEM((2, 1024), jnp.int32),
        pltpu.VMEM((256,), jnp.int32),
        pltpu.VMEM((UNIT,), jnp.float32),
        pltpu.VMEM((UNIT,), jnp.float32),
        pltpu.SemaphoreType.DMA,
        pltpu.SemaphoreType.DMA,
        pltpu.SemaphoreType.DMA,
    ],
    compiler_params=pltpu.CompilerParams(
        use_tc_tiling_on_sc=False, needs_layout_passes=False
    ),
)(_sc_body)


def kernel(atom_indices, valence_configs, core_configs, W_valence, W_core):
    table = _build_table(valence_configs, core_configs, W_valence, W_core)
    idx4 = (
        atom_indices.astype(jnp.int32)
        .reshape(8, 128, L // 8, 8)
        .transpose(2, 0, 3, 1)
    )
    out = _gather_rows(table.reshape(N_ROWS * D), idx4)
    return (
        out.reshape(L, 8, 8, 8, 128)
        .transpose(2, 4, 0, 1, 3)
        .reshape(B, L, D)
    )


# speedup vs baseline: 1.1159x; 1.1159x over previous
import functools

import jax
import jax.numpy as jnp
from jax import lax
from jax.experimental import pallas as pl
from jax.experimental.pallas import tpu as pltpu
from jax.experimental.pallas import tpu_sc as plsc

K = 12
D = 64
N_ROWS = 19
B = 1024
L = 200

NC, NS = 2, 16
NW = NC * NS
L_UNITS_MAX = (L + NW - 1) // NW + 1
UNIT = 2 * 8 * 8 * 128
SLAB = D * B


def _table_body(vc_ref, cc_ref, wv_ref, wc_ref, out_ref):
    zero = jnp.zeros((1, K), jnp.float32)
    aug_v = jnp.concatenate([zero, vc_ref[...]], axis=0)
    aug_c = jnp.concatenate([zero, cc_ref[...]], axis=0)
    tv = jnp.dot(aug_v, wv_ref[...], preferred_element_type=jnp.float32,
                 precision=jax.lax.Precision.HIGHEST)
    tc = jnp.dot(aug_c, wc_ref[...], preferred_element_type=jnp.float32,
                 precision=jax.lax.Precision.HIGHEST)
    out_ref[...] = jnp.concatenate([tv, tc], axis=-1)


def _build_table(vc, cc, wv, wc):
    return pl.pallas_call(
        _table_body,
        out_shape=jax.ShapeDtypeStruct((N_ROWS, D), jnp.float32),
    )(vc, cc, wv, wc)


def _sc_body(table_hbm, idx_hbm, out_hbm,
             table_v, icol, svtab, stag_a, stag_b, sem_a, sem_b, isem):
    w = lax.axis_index("s") * NC + lax.axis_index("c")
    pltpu.sync_copy(table_hbm, table_v)

    iota = lax.iota(jnp.int32, 16)
    for d in range(16):
        cr = (iota + d) & 15
        svtab[pl.ds(d * 16, 16)] = (cr >> 3) * 8192 + (cr & 7) * 128 + iota

    lo = w * 25
    hi = lo + 25
    stags = (stag_a, stag_b)
    sems = (sem_a, sem_b)

    l0 = lo >> 2
    for bb in range(8):
        pltpu.async_copy(
            idx_hbm.at[l0 >> 3, bb, l0 & 7],
            icol.at[l0 & 1, pl.ds(bb * 128, 128)],
            isem,
        )

    def _unit(u, carry):
        l = u >> 2
        cq = u & 3
        isel = l & 1

        @pl.when((cq == 0) | (u == lo))
        def _drain_icol():
            for bb in range(8):
                pltpu.make_async_copy(
                    idx_hbm.at[0, 0, 0],
                    icol.at[isel, pl.ds(bb * 128, 128)],
                    isem,
                ).wait()

        @pl.when(((cq == 3) | (u == hi - 1)) & (u + 1 < hi))
        def _prefetch():
            nxt = l + 1
            for bb in range(8):
                pltpu.async_copy(
                    idx_hbm.at[nxt >> 3, bb, nxt & 7],
                    icol.at[1 - isel, pl.ds(bb * 128, 128)],
                    isem,
                )

        for par in range(2):
            stag = stags[par]
            sem = sems[par]

            @pl.when((u & 1) == par)
            def _do(_stag=stag, _sem=sem, _par=par):
                @pl.when(u >= lo + 2)
                def _drain():
                    pltpu.make_async_copy(out_hbm.at[0], _stag, _sem).wait()

                @plsc.parallel_loop(0, 64, unroll=2)
                def _group(g, _stag=_stag):
                    v = icol[isel, pl.ds(g * 16, 16)]
                    v64 = v * D
                    goff = (g >> 3) * 1024 + (g & 7) * 16
                    tb = cq * 16
                    dg = iota
                    for d in range(16):
                        vals = plsc.load_gather(
                            table_v.at[pl.ds(tb, (N_ROWS - 1) * D + 16)],
                            [v64 + dg],
                        )
                        sv = svtab[pl.ds(d * 16, 16)]
                        plsc.store_scatter(
                            _stag.at[pl.ds(goff, 8192 + 896 + 16)], [sv], vals
                        )
                        dg = (dg + 1) & 15

                pltpu.async_copy(_stag, out_hbm.at[u], _sem)
        return carry

    lax.fori_loop(lo, hi, _unit, 0)
    for par in range(2):
        pltpu.make_async_copy(out_hbm.at[0], stags[par], sems[par]).wait()


_gather_rows = functools.partial(
    pl.kernel,
    out_type=jax.ShapeDtypeStruct((L * 4, UNIT), jnp.float32),
    mesh=plsc.VectorSubcoreMesh(core_axis_name="c", subcore_axis_name="s"),
    scratch_types=[
        pltpu.VMEM((N_ROWS * D,), jnp.float32),
        pltpu.VMEM((2, 1024), jnp.int32),
        pltpu.VMEM((256,), jnp.int32),
        pltpu.VMEM((UNIT,), jnp.float32),
        pltpu.VMEM((UNIT,), jnp.float32),
        pltpu.SemaphoreType.DMA,
        pltpu.SemaphoreType.DMA,
        pltpu.SemaphoreType.DMA,
    ],
    compiler_params=pltpu.CompilerParams(
        use_tc_tiling_on_sc=False, needs_layout_passes=False
    ),
)(_sc_body)


def kernel(atom_indices, valence_configs, core_configs, W_valence, W_core):
    table = _build_table(valence_configs, core_configs, W_valence, W_core)
    idx4 = (
        atom_indices.astype(jnp.int32)
        .reshape(8, 128, L // 8, 8)
        .transpose(2, 0, 3, 1)
    )
    out = _gather_rows(table.reshape(N_ROWS * D), idx4)
    return (
        out.reshape(L, 8, 8, 8, 128)
        .transpose(2, 4, 0, 1, 3)
        .reshape(B, L, D)
    )
